# Initial kernel scaffold; baseline (speedup 1.0000x reference)
import jax
import jax.numpy as jnp
from jax.experimental import pallas as pl


def kernel(graph_nodes, graph_edge_links, graph_edges, mask, params):
    raise NotImplementedError("write your pallas kernel here")


# speedup vs baseline: 2.4745x; 2.4745x over previous
import functools

import jax
import jax.numpy as jnp
from jax import lax
from jax.experimental import pallas as pl
from jax.experimental.pallas import tpu as pltpu
from jax.experimental.pallas import tpu_sc as plsc

_B, _N, _E, _DIN, _H, _DOUT = 8, 1250, 40000, 128, 128, 64
_TOT = _B * _N
_NE = _B * _E
_NC, _NS = 2, 16
_NW = _NC * _NS
_CH = 128
_NE_PAD = -(-_NE // 2048) * 2048
_SCN = _TOT // _NC
_TPN = 320
_ACC = 5120
_RPT = _ACC // _NS
_LMAX = _NE + _CH
_LPAD = -(-_LMAX // _CH) * _CH
_SCAN = 2048


def _leaky(x):
    return jnp.where(x >= 0, x, 0.01 * x)


def _dot(a, b):
    return jnp.dot(a, b, preferred_element_type=jnp.float32)


def _sc_mesh():
    return plsc.VectorSubcoreMesh(
        core_axis_name="c", subcore_axis_name="s", num_cores=_NC, num_subcores=_NS
    )


@functools.lru_cache(maxsize=None)
def _build_partition():
    @functools.partial(
        pl.kernel,
        mesh=_sc_mesh(),
        out_type=(
            jax.ShapeDtypeStruct((_NW, 1, _LPAD), jnp.int32),
            jax.ShapeDtypeStruct((_NW, 1, _LPAD), jnp.int32),
            jax.ShapeDtypeStruct((_NW, 1, 16), jnp.int32),
        ),
        scratch_types=[
            pltpu.VMEM((_SCAN,), jnp.int32),
            pltpu.VMEM((_SCAN,), jnp.int32),
            pltpu.VMEM((_CH + 32,), jnp.int32),
            pltpu.VMEM((_CH + 32,), jnp.int32),
            pltpu.VMEM((16,), jnp.int32),
        ],
        compiler_params=pltpu.CompilerParams(needs_layout_passes=False),
    )
    def _part(src_hbm, dst_hbm, osrc, odst, ocnt, in_s, in_d, st_s, st_d, cnt_v):
        c = lax.axis_index("c")
        s = lax.axis_index("s")
        w = c * _NS + s
        lo = c * _SCN + s * _TPN
        hi = jnp.minimum(lo + _TPN, (c + 1) * _SCN)
        base = c * _SCN

        iota = lax.iota(jnp.int32, 16)

        def _cumsum16(x):
            dnums = lax.GatherDimensionNumbers(
                offset_dims=(), collapsed_slice_dims=(0,), start_index_map=(0,))
            y = x
            for k in (1, 2, 4, 8):
                idxv = jnp.maximum(iota - k, 0)
                sh = lax.gather(y, idxv[:, None], dnums, slice_sizes=(1,),
                                mode=lax.GatherScatterMode.PROMISE_IN_BOUNDS)
                y = y + jnp.where(iota >= k, sh, 0)
            return y

        def _append(sv, dv, mi, off, nf):
            m = mi > 0
            csum = _cumsum16(mi)
            idx = jnp.where(m, off + csum - mi, _CH + 16)
            plsc.store_scatter(st_s, [idx], sv)
            plsc.store_scatter(st_d, [idx], dv)
            off = off + jnp.max(csum, axis=0)
            fl = off >= _CH

            @pl.when(fl)
            def _():
                pltpu.sync_copy(st_s.at[pl.ds(0, _CH)],
                                osrc.at[w, 0, pl.ds(nf * _CH, _CH)])
                pltpu.sync_copy(st_d.at[pl.ds(0, _CH)],
                                odst.at[w, 0, pl.ds(nf * _CH, _CH)])
                st_s[pl.ds(0, 16)] = st_s[pl.ds(_CH, 16)]
                st_d[pl.ds(0, 16)] = st_d[pl.ds(_CH, 16)]

            off = jnp.where(fl, off - _CH, off)
            nf = jnp.where(fl, nf + 1, nf)
            return off, nf

        def _scan_chunk(g, carry):
            off, nf = carry
            pltpu.sync_copy(src_hbm.at[pl.ds(g * _SCAN, _SCAN)], in_s)
            pltpu.sync_copy(dst_hbm.at[pl.ds(g * _SCAN, _SCAN)], in_d)

            def _vreg(j, carry2):
                off2, nf2 = carry2
                sv = in_s[pl.ds(j * 16, 16)]
                dv = in_d[pl.ds(j * 16, 16)]
                mi = jnp.maximum(
                    jnp.minimum(jnp.minimum(dv - (lo - 1), hi - dv), 1), 0)
                return _append(sv, dv - base, mi, off2, nf2)

            return lax.fori_loop(0, _SCAN // 16, _vreg, (off, nf))

        off, nf = lax.fori_loop(0, _NE_PAD // _SCAN, _scan_chunk, (0, 0))

        dsv = jnp.zeros((16,), jnp.int32)
        ddv = jnp.broadcast_to(_SCN + s, (16,))
        ones = dsv + 1
        for _ in range(_CH // 16):
            off, nf = _append(dsv, ddv, ones, off, nf)

        cnt_v[...] = jnp.broadcast_to(nf, (16,))
        pltpu.sync_copy(cnt_v, ocnt.at[w, 0])

    return _part


def _partition_edges(src, dst):
    return _build_partition()(src, dst)


@functools.lru_cache(maxsize=None)
def _build_sc_segment_sum():
    @functools.partial(
        pl.kernel,
        mesh=_sc_mesh(),
        out_type=jax.ShapeDtypeStruct((_NC, _ACC, _H), jnp.float32),
        scratch_types=[
            pltpu.VMEM((16,), jnp.int32),
            pltpu.VMEM((_CH,), jnp.int32),
            pltpu.VMEM((_CH,), jnp.int32),
            pltpu.VMEM((_CH, _H), jnp.float32),
            pltpu.VMEM_SHARED((_ACC, _H), jnp.float32),
            pltpu.SemaphoreType.DMA,
        ],
        compiler_params=pltpu.CompilerParams(needs_layout_passes=False),
    )
    def _seg(x_hbm, osrc, odst, ocnt, out_hbm, cnt_v, src_v, dst_v, rows_v, agg_sh, sem):
        c = lax.axis_index("c")
        s = lax.axis_index("s")
        w = c * _NS + s

        zv = jnp.zeros((16,), jnp.float32)

        def _zrow(i, carry):
            def _zcol(j, carry2):
                rows_v[i, pl.ds(j * 16, 16)] = zv
                return carry2

            return lax.fori_loop(0, _H // 16, _zcol, carry)

        lax.fori_loop(0, _CH, _zrow, 0)
        pltpu.sync_copy(rows_v, agg_sh.at[pl.ds(s * _RPT, _CH)])
        pltpu.sync_copy(rows_v, agg_sh.at[pl.ds(s * _RPT + _CH, _CH)])
        pltpu.sync_copy(rows_v.at[pl.ds(0, _RPT - 2 * _CH)],
                        agg_sh.at[pl.ds(s * _RPT + 2 * _CH, _RPT - 2 * _CH)])
        plsc.subcore_barrier()

        pltpu.sync_copy(ocnt.at[w, 0], cnt_v)
        nch = jnp.max(cnt_v[...], axis=0)

        def _body(i, carry):
            pltpu.sync_copy(osrc.at[w, 0, pl.ds(i * _CH, _CH)], src_v)
            pltpu.sync_copy(odst.at[w, 0, pl.ds(i * _CH, _CH)], dst_v)
            pltpu.async_copy(x_hbm.at[src_v], rows_v, sem).wait()
            pltpu.sync_copy(rows_v, agg_sh.at[dst_v], add=True)
            return carry

        lax.fori_loop(0, nch, _body, 0)

        plsc.subcore_barrier()
        pltpu.sync_copy(
            agg_sh.at[pl.ds(s * _RPT, _RPT)], out_hbm.at[c, pl.ds(s * _RPT, _RPT)]
        )

    return _seg


def _sc_segment_sum(x, osrc, odst, ocnt):
    return _build_sc_segment_sum()(x, osrc, odst, ocnt)


def _conv_body(x_ref, agg_ref, e_ref, w1_ref, b1_ref, w2_ref, b2_ref, o_ref):
    h = e_ref[...] * x_ref[...] + agg_ref[0]
    h = _leaky(_dot(h, w1_ref[...]) + b1_ref[...])
    h = _leaky(_dot(h, w2_ref[...]) + b2_ref[...])
    o_ref[...] = _leaky(h)


_CROWS = 1000


def _tc_conv(x, agg, epsv, w1, b1, w2, b2):
    grid = (_TOT // _CROWS,)
    return pl.pallas_call(
        _conv_body,
        grid=grid,
        in_specs=[
            pl.BlockSpec((_CROWS, _H), lambda i: (i, 0)),
            pl.BlockSpec((1, _CROWS, _H), lambda i: (i // 5, i % 5, 0)),
            pl.BlockSpec((1, _H), lambda i: (0, 0)),
            pl.BlockSpec((_H, _H), lambda i: (0, 0)),
            pl.BlockSpec((1, _H), lambda i: (0, 0)),
            pl.BlockSpec((_H, _H), lambda i: (0, 0)),
            pl.BlockSpec((1, _H), lambda i: (0, 0)),
        ],
        out_specs=pl.BlockSpec((_CROWS, _H), lambda i: (i, 0)),
        out_shape=jax.ShapeDtypeStruct((_TOT, _H), jnp.float32),
    )(x, agg, epsv, w1, b1, w2, b2)


def _head_body(x_ref, m_ref, w1_ref, b1_ref, w2_ref, b2_ref, w3_ref, b3_ref, o_ref):
    xb = x_ref[0]
    h = _leaky(_dot(xb, w1_ref[...]) + b1_ref[...])
    h = _leaky(_dot(h, w2_ref[...]) + b2_ref[...])
    z = _dot(h, w3_ref[...]) + b3_ref[...]
    z = jnp.where(m_ref[0] != 0, z, -jnp.inf)
    zm = jnp.max(z, axis=0, keepdims=True)
    ez = jnp.exp(z - zm)
    o_ref[0] = ez / jnp.sum(ez, axis=0, keepdims=True)


def _tc_head(x3, m3, w1, b1, w2, b2, w3, b3):
    return pl.pallas_call(
        _head_body,
        grid=(_B,),
        in_specs=[
            pl.BlockSpec((1, _N, _H), lambda b: (b, 0, 0)),
            pl.BlockSpec((1, _N, 1), lambda b: (b, 0, 0)),
            pl.BlockSpec((_H, _H), lambda b: (0, 0)),
            pl.BlockSpec((1, _H), lambda b: (0, 0)),
            pl.BlockSpec((_H, _DOUT), lambda b: (0, 0)),
            pl.BlockSpec((1, _DOUT), lambda b: (0, 0)),
            pl.BlockSpec((_DOUT, 1), lambda b: (0, 0)),
            pl.BlockSpec((1, 1), lambda b: (0, 0)),
        ],
        out_specs=pl.BlockSpec((1, _N, 1), lambda b: (b, 0, 0)),
        out_shape=jax.ShapeDtypeStruct((_B, _N, 1), jnp.float32),
    )(x3, m3, w1, b1, w2, b2, w3, b3)


def kernel(graph_nodes, graph_edge_links, graph_edges, mask, params):
    del graph_edges
    links = graph_edge_links.astype(jnp.int32)
    offs = (jnp.arange(_B, dtype=jnp.int32) * _N)[:, None]
    src = (links[:, 0, :] + offs).reshape(-1)
    dst = (links[:, 1, :] + offs).reshape(-1)
    npad = _NE_PAD - _NE
    src = jnp.concatenate([src, jnp.zeros((npad,), jnp.int32)])
    dst = jnp.concatenate([dst, jnp.full((npad,), _TOT, jnp.int32)])
    osrc, odst, ocnt = _partition_edges(src, dst)

    x = graph_nodes.reshape(_TOT, _DIN)
    for li in range(4):
        p = params["conv%d" % li]
        agg = _sc_segment_sum(x, osrc, odst, ocnt)
        epsv = jnp.full((1, _H), 1.0, jnp.float32) + p["eps"]
        x = _tc_conv(x, agg, epsv, p["W1"], p["b1"].reshape(1, _H),
                     p["W2"], p["b2"].reshape(1, _H))

    m3 = mask.astype(jnp.int32).reshape(_B, _N, 1)
    probs = _tc_head(
        x.reshape(_B, _N, _H), m3,
        params["lin1"]["W"], params["lin1"]["b"].reshape(1, _H),
        params["lin2"]["W"], params["lin2"]["b"].reshape(1, _DOUT),
        params["lin3"]["W"], params["lin3"]["b"].reshape(1, 1),
    )
    return probs.reshape(_B, _N)


# speedup vs baseline: 2.7035x; 1.0926x over previous
import functools

import jax
import jax.numpy as jnp
from jax import lax
from jax.experimental import pallas as pl
from jax.experimental.pallas import tpu as pltpu
from jax.experimental.pallas import tpu_sc as plsc

_B, _N, _E, _DIN, _H, _DOUT = 8, 1250, 40000, 128, 128, 64
_TOT = _B * _N
_NE = _B * _E
_NC, _NS = 2, 16
_NW = _NC * _NS
_CH = 128
_NE_PAD = -(-_NE // 2048) * 2048
_SCN = _TOT // _NC
_TPN = 320
_ACC = 5120
_RPT = _ACC // _NS
_LMAX = _NE + _CH
_LPAD = -(-_LMAX // _CH) * _CH
_SCAN = 2048


def _leaky(x):
    return jnp.where(x >= 0, x, 0.01 * x)


def _dot(a, b):
    return jnp.dot(a, b, preferred_element_type=jnp.float32)


def _sc_mesh():
    return plsc.VectorSubcoreMesh(
        core_axis_name="c", subcore_axis_name="s", num_cores=_NC, num_subcores=_NS
    )


@functools.lru_cache(maxsize=None)
def _build_partition():
    @functools.partial(
        pl.kernel,
        mesh=_sc_mesh(),
        out_type=(
            jax.ShapeDtypeStruct((_NW, 1, _LPAD), jnp.int32),
            jax.ShapeDtypeStruct((_NW, 1, _LPAD), jnp.int32),
            jax.ShapeDtypeStruct((_NW, 1, 16), jnp.int32),
        ),
        scratch_types=[
            pltpu.VMEM((_SCAN,), jnp.int32),
            pltpu.VMEM((_SCAN,), jnp.int32),
            pltpu.VMEM((_CH + 32,), jnp.int32),
            pltpu.VMEM((_CH + 32,), jnp.int32),
            pltpu.VMEM((16,), jnp.int32),
        ],
        compiler_params=pltpu.CompilerParams(needs_layout_passes=False),
    )
    def _part(src_hbm, dst_hbm, osrc, odst, ocnt, in_s, in_d, st_s, st_d, cnt_v):
        c = lax.axis_index("c")
        s = lax.axis_index("s")
        w = c * _NS + s
        lo = c * _SCN + s * _TPN
        hi = jnp.minimum(lo + _TPN, (c + 1) * _SCN)
        base = c * _SCN

        def _append(sv, dv, mi, off, nf):
            m = mi > 0
            csum = plsc.cumsum(mi)
            idx = jnp.where(m, off + csum - mi, _CH + 16)
            plsc.store_scatter(st_s, [idx], sv)
            plsc.store_scatter(st_d, [idx], dv)
            off = off + csum[15]
            fl = off >= _CH

            @pl.when(fl)
            def _():
                pltpu.sync_copy(st_s.at[pl.ds(0, _CH)],
                                osrc.at[w, 0, pl.ds(nf * _CH, _CH)])
                pltpu.sync_copy(st_d.at[pl.ds(0, _CH)],
                                odst.at[w, 0, pl.ds(nf * _CH, _CH)])
                st_s[pl.ds(0, 16)] = st_s[pl.ds(_CH, 16)]
                st_d[pl.ds(0, 16)] = st_d[pl.ds(_CH, 16)]

            off = jnp.where(fl, off - _CH, off)
            nf = jnp.where(fl, nf + 1, nf)
            return off, nf

        def _scan_chunk(g, carry):
            off, nf = carry
            pltpu.sync_copy(src_hbm.at[pl.ds(g * _SCAN, _SCAN)], in_s)
            pltpu.sync_copy(dst_hbm.at[pl.ds(g * _SCAN, _SCAN)], in_d)

            def _vreg(j, carry2):
                off2, nf2 = carry2
                sv = in_s[pl.ds(j * 16, 16)]
                dv = in_d[pl.ds(j * 16, 16)]
                mi = jnp.maximum(
                    jnp.minimum(jnp.minimum(dv - (lo - 1), hi - dv), 1), 0)
                return _append(sv, dv - base, mi, off2, nf2)

            return lax.fori_loop(0, _SCAN // 16, _vreg, (off, nf))

        off, nf = lax.fori_loop(0, _NE_PAD // _SCAN, _scan_chunk, (0, 0))

        dsv = jnp.zeros((16,), jnp.int32)
        ddv = jnp.broadcast_to(_SCN + s, (16,))
        ones = dsv + 1
        for _ in range(_CH // 16):
            off, nf = _append(dsv, ddv, ones, off, nf)

        cnt_v[...] = jnp.broadcast_to(nf, (16,))
        pltpu.sync_copy(cnt_v, ocnt.at[w, 0])

    return _part


def _partition_edges(src, dst):
    return _build_partition()(src, dst)


@functools.lru_cache(maxsize=None)
def _build_sc_segment_sum():
    @functools.partial(
        pl.kernel,
        mesh=_sc_mesh(),
        out_type=jax.ShapeDtypeStruct((_NC, _ACC, _H), jnp.float32),
        scratch_types=[
            pltpu.VMEM((16,), jnp.int32),
            pltpu.VMEM((_CH,), jnp.int32),
            pltpu.VMEM((_CH,), jnp.int32),
            pltpu.VMEM((_CH, _H), jnp.float32),
            pltpu.VMEM_SHARED((_ACC, _H), jnp.float32),
            pltpu.SemaphoreType.DMA,
        ],
        compiler_params=pltpu.CompilerParams(needs_layout_passes=False),
    )
    def _seg(x_hbm, osrc, odst, ocnt, out_hbm, cnt_v, src_v, dst_v, rows_v, agg_sh, sem):
        c = lax.axis_index("c")
        s = lax.axis_index("s")
        w = c * _NS + s

        zv = jnp.zeros((16,), jnp.float32)

        def _zrow(i, carry):
            def _zcol(j, carry2):
                rows_v[i, pl.ds(j * 16, 16)] = zv
                return carry2

            return lax.fori_loop(0, _H // 16, _zcol, carry)

        lax.fori_loop(0, _CH, _zrow, 0)
        pltpu.sync_copy(rows_v, agg_sh.at[pl.ds(s * _RPT, _CH)])
        pltpu.sync_copy(rows_v, agg_sh.at[pl.ds(s * _RPT + _CH, _CH)])
        pltpu.sync_copy(rows_v.at[pl.ds(0, _RPT - 2 * _CH)],
                        agg_sh.at[pl.ds(s * _RPT + 2 * _CH, _RPT - 2 * _CH)])
        plsc.subcore_barrier()

        pltpu.sync_copy(ocnt.at[w, 0], cnt_v)
        nch = jnp.max(cnt_v[...], axis=0)

        def _body(i, carry):
            pltpu.sync_copy(osrc.at[w, 0, pl.ds(i * _CH, _CH)], src_v)
            pltpu.sync_copy(odst.at[w, 0, pl.ds(i * _CH, _CH)], dst_v)
            pltpu.async_copy(x_hbm.at[src_v], rows_v, sem).wait()
            pltpu.sync_copy(rows_v, agg_sh.at[dst_v], add=True)
            return carry

        lax.fori_loop(0, nch, _body, 0)

        plsc.subcore_barrier()
        pltpu.sync_copy(
            agg_sh.at[pl.ds(s * _RPT, _RPT)], out_hbm.at[c, pl.ds(s * _RPT, _RPT)]
        )

    return _seg


def _sc_segment_sum(x, osrc, odst, ocnt):
    return _build_sc_segment_sum()(x, osrc, odst, ocnt)


def _conv_body(x_ref, agg_ref, e_ref, w1_ref, b1_ref, w2_ref, b2_ref, o_ref):
    h = e_ref[...] * x_ref[...] + agg_ref[0]
    h = _leaky(_dot(h, w1_ref[...]) + b1_ref[...])
    h = _leaky(_dot(h, w2_ref[...]) + b2_ref[...])
    o_ref[...] = _leaky(h)


_CROWS = 1000


def _tc_conv(x, agg, epsv, w1, b1, w2, b2):
    grid = (_TOT // _CROWS,)
    return pl.pallas_call(
        _conv_body,
        grid=grid,
        in_specs=[
            pl.BlockSpec((_CROWS, _H), lambda i: (i, 0)),
            pl.BlockSpec((1, _CROWS, _H), lambda i: (i // 5, i % 5, 0)),
            pl.BlockSpec((1, _H), lambda i: (0, 0)),
            pl.BlockSpec((_H, _H), lambda i: (0, 0)),
            pl.BlockSpec((1, _H), lambda i: (0, 0)),
            pl.BlockSpec((_H, _H), lambda i: (0, 0)),
            pl.BlockSpec((1, _H), lambda i: (0, 0)),
        ],
        out_specs=pl.BlockSpec((_CROWS, _H), lambda i: (i, 0)),
        out_shape=jax.ShapeDtypeStruct((_TOT, _H), jnp.float32),
    )(x, agg, epsv, w1, b1, w2, b2)


def _head_body(x_ref, m_ref, w1_ref, b1_ref, w2_ref, b2_ref, w3_ref, b3_ref, o_ref):
    xb = x_ref[0]
    h = _leaky(_dot(xb, w1_ref[...]) + b1_ref[...])
    h = _leaky(_dot(h, w2_ref[...]) + b2_ref[...])
    z = _dot(h, w3_ref[...]) + b3_ref[...]
    z = jnp.where(m_ref[0] != 0, z, -jnp.inf)
    zm = jnp.max(z, axis=0, keepdims=True)
    ez = jnp.exp(z - zm)
    o_ref[0] = ez / jnp.sum(ez, axis=0, keepdims=True)


def _tc_head(x3, m3, w1, b1, w2, b2, w3, b3):
    return pl.pallas_call(
        _head_body,
        grid=(_B,),
        in_specs=[
            pl.BlockSpec((1, _N, _H), lambda b: (b, 0, 0)),
            pl.BlockSpec((1, _N, 1), lambda b: (b, 0, 0)),
            pl.BlockSpec((_H, _H), lambda b: (0, 0)),
            pl.BlockSpec((1, _H), lambda b: (0, 0)),
            pl.BlockSpec((_H, _DOUT), lambda b: (0, 0)),
            pl.BlockSpec((1, _DOUT), lambda b: (0, 0)),
            pl.BlockSpec((_DOUT, 1), lambda b: (0, 0)),
            pl.BlockSpec((1, 1), lambda b: (0, 0)),
        ],
        out_specs=pl.BlockSpec((1, _N, 1), lambda b: (b, 0, 0)),
        out_shape=jax.ShapeDtypeStruct((_B, _N, 1), jnp.float32),
    )(x3, m3, w1, b1, w2, b2, w3, b3)


def kernel(graph_nodes, graph_edge_links, graph_edges, mask, params):
    del graph_edges
    links = graph_edge_links.astype(jnp.int32)
    offs = (jnp.arange(_B, dtype=jnp.int32) * _N)[:, None]
    src = (links[:, 0, :] + offs).reshape(-1)
    dst = (links[:, 1, :] + offs).reshape(-1)
    npad = _NE_PAD - _NE
    src = jnp.concatenate([src, jnp.zeros((npad,), jnp.int32)])
    dst = jnp.concatenate([dst, jnp.full((npad,), _TOT, jnp.int32)])
    osrc, odst, ocnt = _partition_edges(src, dst)

    x = graph_nodes.reshape(_TOT, _DIN)
    for li in range(4):
        p = params["conv%d" % li]
        agg = _sc_segment_sum(x, osrc, odst, ocnt)
        epsv = jnp.full((1, _H), 1.0, jnp.float32) + p["eps"]
        x = _tc_conv(x, agg, epsv, p["W1"], p["b1"].reshape(1, _H),
                     p["W2"], p["b2"].reshape(1, _H))

    m3 = mask.astype(jnp.int32).reshape(_B, _N, 1)
    probs = _tc_head(
        x.reshape(_B, _N, _H), m3,
        params["lin1"]["W"], params["lin1"]["b"].reshape(1, _H),
        params["lin2"]["W"], params["lin2"]["b"].reshape(1, _DOUT),
        params["lin3"]["W"], params["lin3"]["b"].reshape(1, 1),
    )
    return probs.reshape(_B, _N)


# speedup vs baseline: 3.1755x; 1.1746x over previous
import functools

import jax
import jax.numpy as jnp
from jax import lax
from jax.experimental import pallas as pl
from jax.experimental.pallas import tpu as pltpu
from jax.experimental.pallas import tpu_sc as plsc

_B, _N, _E, _DIN, _H, _DOUT = 8, 1250, 40000, 128, 128, 64
_TOT = _B * _N
_NE = _B * _E
_NC, _NS = 2, 16
_NW = _NC * _NS
_CH = 128
_NE_PAD = -(-_NE // 2048) * 2048
_SCN = _TOT // _NC
_TPN = 320
_ACC = 5120
_RPT = _ACC // _NS
_LMAX = _NE + _CH
_LPAD = -(-_LMAX // _CH) * _CH
_SCAN = 2048


def _leaky(x):
    return jnp.where(x >= 0, x, 0.01 * x)


def _dot(a, b):
    return jnp.dot(a, b, preferred_element_type=jnp.float32)


def _sc_mesh():
    return plsc.VectorSubcoreMesh(
        core_axis_name="c", subcore_axis_name="s", num_cores=_NC, num_subcores=_NS
    )


@functools.lru_cache(maxsize=None)
def _build_partition():
    @functools.partial(
        pl.kernel,
        mesh=_sc_mesh(),
        out_type=(
            jax.ShapeDtypeStruct((_NW, 1, _LPAD), jnp.int32),
            jax.ShapeDtypeStruct((_NW, 1, _LPAD), jnp.int32),
            jax.ShapeDtypeStruct((_NW, 1, 16), jnp.int32),
        ),
        scratch_types=[
            pltpu.VMEM((_SCAN,), jnp.int32),
            pltpu.VMEM((_SCAN,), jnp.int32),
            pltpu.VMEM((_CH + 32,), jnp.int32),
            pltpu.VMEM((_CH + 32,), jnp.int32),
            pltpu.VMEM((16,), jnp.int32),
        ],
        compiler_params=pltpu.CompilerParams(needs_layout_passes=False),
    )
    def _part(src_hbm, dst_hbm, osrc, odst, ocnt, in_s, in_d, st_s, st_d, cnt_v):
        c = lax.axis_index("c")
        s = lax.axis_index("s")
        w = c * _NS + s
        lo = c * _SCN + s * _TPN
        hi = jnp.minimum(lo + _TPN, (c + 1) * _SCN)
        base = c * _SCN

        def _append(sv, dv, mi, off, nf):
            m = mi > 0
            csum = plsc.cumsum(mi)
            idx = jnp.where(m, off + csum - mi, _CH + 16)
            plsc.store_scatter(st_s, [idx], sv)
            plsc.store_scatter(st_d, [idx], dv)
            off = off + csum[15]
            fl = off >= _CH

            @pl.when(fl)
            def _():
                pltpu.sync_copy(st_s.at[pl.ds(0, _CH)],
                                osrc.at[w, 0, pl.ds(nf * _CH, _CH)])
                pltpu.sync_copy(st_d.at[pl.ds(0, _CH)],
                                odst.at[w, 0, pl.ds(nf * _CH, _CH)])
                st_s[pl.ds(0, 16)] = st_s[pl.ds(_CH, 16)]
                st_d[pl.ds(0, 16)] = st_d[pl.ds(_CH, 16)]

            off = jnp.where(fl, off - _CH, off)
            nf = jnp.where(fl, nf + 1, nf)
            return off, nf

        def _scan_chunk(g, carry):
            off, nf = carry
            pltpu.sync_copy(src_hbm.at[pl.ds(g * _SCAN, _SCAN)], in_s)
            pltpu.sync_copy(dst_hbm.at[pl.ds(g * _SCAN, _SCAN)], in_d)

            def _vreg(j, carry2):
                off2, nf2 = carry2
                sv = in_s[pl.ds(j * 16, 16)]
                dv = in_d[pl.ds(j * 16, 16)]
                mi = jnp.maximum(
                    jnp.minimum(jnp.minimum(dv - (lo - 1), hi - dv), 1), 0)
                return _append(sv, dv - base, mi, off2, nf2)

            return lax.fori_loop(0, _SCAN // 16, _vreg, (off, nf))

        off, nf = lax.fori_loop(0, _NE_PAD // _SCAN, _scan_chunk, (0, 0))

        dsv = jnp.zeros((16,), jnp.int32)
        ddv = jnp.broadcast_to(_SCN + s, (16,))
        ones = dsv + 1
        for _ in range(_CH // 16):
            off, nf = _append(dsv, ddv, ones, off, nf)

        cnt_v[...] = jnp.broadcast_to(nf, (16,))
        pltpu.sync_copy(cnt_v, ocnt.at[w, 0])

    return _part


def _partition_edges(src, dst):
    return _build_partition()(src, dst)


@functools.lru_cache(maxsize=None)
def _build_sc_segment_sum():
    @functools.partial(
        pl.kernel,
        mesh=_sc_mesh(),
        out_type=jax.ShapeDtypeStruct((_NC, _ACC, _H), jnp.float32),
        scratch_types=[
            pltpu.VMEM((16,), jnp.int32),
            pltpu.VMEM((_CH,), jnp.int32),
            pltpu.VMEM((_CH,), jnp.int32),
            pltpu.VMEM((_CH,), jnp.int32),
            pltpu.VMEM((_CH,), jnp.int32),
            pltpu.VMEM((_CH, _H), jnp.float32),
            pltpu.VMEM((_CH, _H), jnp.float32),
            pltpu.VMEM_SHARED((_ACC, _H), jnp.float32),
            pltpu.SemaphoreType.DMA,
            pltpu.SemaphoreType.DMA,
        ],
        compiler_params=pltpu.CompilerParams(needs_layout_passes=False),
    )
    def _seg(x_hbm, osrc, odst, ocnt, out_hbm, cnt_v, src_a, src_b, dst_a,
             dst_b, rows_a, rows_b, agg_sh, sem_a, sem_b):
        c = lax.axis_index("c")
        s = lax.axis_index("s")
        w = c * _NS + s

        zv = jnp.zeros((16,), jnp.float32)

        def _zrow(i, carry):
            def _zcol(j, carry2):
                rows_a[i, pl.ds(j * 16, 16)] = zv
                return carry2

            return lax.fori_loop(0, _H // 16, _zcol, carry)

        lax.fori_loop(0, _CH, _zrow, 0)
        pltpu.sync_copy(rows_a, agg_sh.at[pl.ds(s * _RPT, _CH)])
        pltpu.sync_copy(rows_a, agg_sh.at[pl.ds(s * _RPT + _CH, _CH)])
        pltpu.sync_copy(rows_a.at[pl.ds(0, _RPT - 2 * _CH)],
                        agg_sh.at[pl.ds(s * _RPT + 2 * _CH, _RPT - 2 * _CH)])
        plsc.subcore_barrier()

        pltpu.sync_copy(ocnt.at[w, 0], cnt_v)
        nch = jnp.max(cnt_v[...], axis=0)

        def _issue(i, src_x, rows_x, sem_x):
            pltpu.sync_copy(osrc.at[w, 0, pl.ds(i * _CH, _CH)], src_x)
            pltpu.async_copy(x_hbm.at[src_x], rows_x, sem_x)

        @pl.when(nch > 0)
        def _():
            _issue(0, src_a, rows_a, sem_a)

        def _consume(i, src_x, rows_x, sem_x, dst_x, src_y, rows_y, sem_y):
            pltpu.make_async_copy(x_hbm.at[src_x], rows_x, sem_x).wait()

            @pl.when(i + 1 < nch)
            def _():
                _issue(i + 1, src_y, rows_y, sem_y)

            pltpu.sync_copy(odst.at[w, 0, pl.ds(i * _CH, _CH)], dst_x)
            pltpu.sync_copy(rows_x, agg_sh.at[dst_x], add=True)

        def _body(i, carry):
            even = lax.rem(i, 2) == 0

            @pl.when(even)
            def _():
                _consume(i, src_a, rows_a, sem_a, dst_a, src_b, rows_b, sem_b)

            @pl.when(jnp.logical_not(even))
            def _():
                _consume(i, src_b, rows_b, sem_b, dst_b, src_a, rows_a, sem_a)

            return carry

        lax.fori_loop(0, nch, _body, 0)

        plsc.subcore_barrier()
        pltpu.sync_copy(
            agg_sh.at[pl.ds(s * _RPT, _RPT)], out_hbm.at[c, pl.ds(s * _RPT, _RPT)]
        )

    return _seg


def _sc_segment_sum(x, osrc, odst, ocnt):
    return _build_sc_segment_sum()(x, osrc, odst, ocnt)


def _conv_body(x_ref, agg_ref, e_ref, w1_ref, b1_ref, w2_ref, b2_ref, o_ref):
    h = e_ref[...] * x_ref[...] + agg_ref[0]
    h = _leaky(_dot(h, w1_ref[...]) + b1_ref[...])
    h = _leaky(_dot(h, w2_ref[...]) + b2_ref[...])
    o_ref[...] = _leaky(h)


_CROWS = 1000


def _tc_conv(x, agg, epsv, w1, b1, w2, b2):
    grid = (_TOT // _CROWS,)
    return pl.pallas_call(
        _conv_body,
        grid=grid,
        in_specs=[
            pl.BlockSpec((_CROWS, _H), lambda i: (i, 0)),
            pl.BlockSpec((1, _CROWS, _H), lambda i: (i // 5, i % 5, 0)),
            pl.BlockSpec((1, _H), lambda i: (0, 0)),
            pl.BlockSpec((_H, _H), lambda i: (0, 0)),
            pl.BlockSpec((1, _H), lambda i: (0, 0)),
            pl.BlockSpec((_H, _H), lambda i: (0, 0)),
            pl.BlockSpec((1, _H), lambda i: (0, 0)),
        ],
        out_specs=pl.BlockSpec((_CROWS, _H), lambda i: (i, 0)),
        out_shape=jax.ShapeDtypeStruct((_TOT, _H), jnp.float32),
    )(x, agg, epsv, w1, b1, w2, b2)


def _head_body(x_ref, m_ref, w1_ref, b1_ref, w2_ref, b2_ref, w3_ref, b3_ref, o_ref):
    xb = x_ref[0]
    h = _leaky(_dot(xb, w1_ref[...]) + b1_ref[...])
    h = _leaky(_dot(h, w2_ref[...]) + b2_ref[...])
    z = _dot(h, w3_ref[...]) + b3_ref[...]
    z = jnp.where(m_ref[0] != 0, z, -jnp.inf)
    zm = jnp.max(z, axis=0, keepdims=True)
    ez = jnp.exp(z - zm)
    o_ref[0] = ez / jnp.sum(ez, axis=0, keepdims=True)


def _tc_head(x3, m3, w1, b1, w2, b2, w3, b3):
    return pl.pallas_call(
        _head_body,
        grid=(_B,),
        in_specs=[
            pl.BlockSpec((1, _N, _H), lambda b: (b, 0, 0)),
            pl.BlockSpec((1, _N, 1), lambda b: (b, 0, 0)),
            pl.BlockSpec((_H, _H), lambda b: (0, 0)),
            pl.BlockSpec((1, _H), lambda b: (0, 0)),
            pl.BlockSpec((_H, _DOUT), lambda b: (0, 0)),
            pl.BlockSpec((1, _DOUT), lambda b: (0, 0)),
            pl.BlockSpec((_DOUT, 1), lambda b: (0, 0)),
            pl.BlockSpec((1, 1), lambda b: (0, 0)),
        ],
        out_specs=pl.BlockSpec((1, _N, 1), lambda b: (b, 0, 0)),
        out_shape=jax.ShapeDtypeStruct((_B, _N, 1), jnp.float32),
    )(x3, m3, w1, b1, w2, b2, w3, b3)


def kernel(graph_nodes, graph_edge_links, graph_edges, mask, params):
    del graph_edges
    links = graph_edge_links.astype(jnp.int32)
    offs = (jnp.arange(_B, dtype=jnp.int32) * _N)[:, None]
    src = (links[:, 0, :] + offs).reshape(-1)
    dst = (links[:, 1, :] + offs).reshape(-1)
    npad = _NE_PAD - _NE
    src = jnp.concatenate([src, jnp.zeros((npad,), jnp.int32)])
    dst = jnp.concatenate([dst, jnp.full((npad,), _TOT, jnp.int32)])
    osrc, odst, ocnt = _partition_edges(src, dst)

    x = graph_nodes.reshape(_TOT, _DIN)
    for li in range(4):
        p = params["conv%d" % li]
        agg = _sc_segment_sum(x, osrc, odst, ocnt)
        epsv = jnp.full((1, _H), 1.0, jnp.float32) + p["eps"]
        x = _tc_conv(x, agg, epsv, p["W1"], p["b1"].reshape(1, _H),
                     p["W2"], p["b2"].reshape(1, _H))

    m3 = mask.astype(jnp.int32).reshape(_B, _N, 1)
    probs = _tc_head(
        x.reshape(_B, _N, _H), m3,
        params["lin1"]["W"], params["lin1"]["b"].reshape(1, _H),
        params["lin2"]["W"], params["lin2"]["b"].reshape(1, _DOUT),
        params["lin3"]["W"], params["lin3"]["b"].reshape(1, 1),
    )
    return probs.reshape(_B, _N)
